# baseline (speedup 1.0000x reference)
import functools

import jax
import jax.numpy as jnp
import numpy as np
from jax import lax
from jax.experimental import pallas as pl
from jax.experimental.pallas import tpu as pltpu
from jax.experimental.pallas import tpu_sc as plsc

_B, _N, _K = 4, 50000, 16
_NK = _N * _K
_M = _B * _NK
_P = _B * _N
_CH = 128
_NCHUNK = _M // _CH
_CPB = _NK // _CH
_NW = 32
_ROUNDS = -(-_NCHUNK // _NW)

_RB = 320
_PB = _RB // 2
_NBLK = (_M // 8) // _RB
_FB = 1600
_NFB = _P // _FB

_XSLOT = (11, 12, 13, 14, 15, 0, 1, 2)
_EPS = 1e-5
_PREC = jax.lax.Precision.HIGHEST


def _sc_gather_body(table_hbm, idx_hbm, out_hbm, idx_v, rows_v, sem):
    wid = lax.axis_index("s") * 2 + lax.axis_index("c")

    def body(r, _):
        c = wid + r * _NW

        @pl.when(c < _NCHUNK)
        def _():
            pltpu.sync_copy(idx_hbm.at[pl.ds(c * _CH, _CH)], idx_v)
            b_off = (c // _CPB) * _N
            for t in range(_CH // 16):
                sl = pl.ds(t * 16, 16)
                idx_v[sl] = (idx_v[sl] + b_off) * 8
            pltpu.async_copy(table_hbm.at[idx_v], rows_v, sem).wait()
            pltpu.sync_copy(rows_v, out_hbm.at[pl.ds(c * _CH, _CH)])

        return ()

    lax.fori_loop(0, _ROUNDS, body, ())


@functools.lru_cache(maxsize=1)
def _get_sc_gather():
    @functools.partial(
        pl.kernel,
        out_type=jax.ShapeDtypeStruct((_M, 16), jnp.float32),
        mesh=plsc.VectorSubcoreMesh(core_axis_name="c", subcore_axis_name="s"),
        compiler_params=pltpu.CompilerParams(use_tc_tiling_on_sc=False),
        scratch_types=[
            pltpu.VMEM((_CH,), jnp.int32),
            pltpu.VMEM((_CH, 16), jnp.float32),
            pltpu.SemaphoreType.DMA,
        ],
    )
    def _sc_gather(table_hbm, idx_hbm, out_hbm, idx_v, rows_v, sem):
        _sc_gather_body(table_hbm, idx_hbm, out_hbm, idx_v, rows_v, sem)

    return _sc_gather


def _gatherv(table128, neigh_idx):
    out = _get_sc_gather()(table128.reshape(8 * _P, 16), neigh_idx.reshape(_M))
    return out.reshape(_M // 8, 128)


def _affine_from_in_moments(W, b, g, be, s1, m2, cnt):
    mu = s1 / cnt
    cov = m2 / cnt - jnp.outer(mu, mu)
    mean = W @ mu + b
    var = jnp.einsum('oi,ij,oj->o', W, cov, W)
    s = g / jnp.sqrt(var + _EPS)
    return W * s[:, None], s * (b - mean) + be


def _affine_from_out_moments(g, be, s1, m2d, cnt):
    mean = s1 / cnt
    var = m2d / cnt - mean * mean
    s = g / jnp.sqrt(var + _EPS)
    return s, be - mean * s


def _bdiag_j(mat16):
    e = jnp.zeros((128, 128), jnp.float32)
    for i in range(8):
        e = e.at[i * 16:(i + 1) * 16, i * 16:(i + 1) * 16].set(mat16)
    return e


def _rowpat(vec16):
    return jnp.tile(vec16, 8)[None, :].repeat(8, 0)


def _mom_feat_kernel(ftr_ref, out_ref):
    x = ftr_ref[...]
    a = jnp.concatenate(
        [x, jnp.ones((_FB, 1), jnp.float32), jnp.zeros((_FB, 7), jnp.float32)],
        axis=1)
    m = lax.dot_general(a, a, (((0,), (0,)), ((), ())),
                        preferred_element_type=jnp.float32, precision=_PREC)

    @pl.when(pl.program_id(0) == 0)
    def _():
        out_ref[...] = jnp.zeros_like(out_ref)

    out_ref[...] += m


def _mom_feat(ftr):
    return pl.pallas_call(
        _mom_feat_kernel,
        out_shape=jax.ShapeDtypeStruct((16, 16), jnp.float32),
        grid=(_NFB,),
        in_specs=[pl.BlockSpec((_FB, 8), lambda j: (j, 0))],
        out_specs=pl.BlockSpec((16, 16), lambda j: (0, 0)),
    )(ftr)


def _table1_kernel(ftr_ref, xyz_ref, wp_ref, out_ref):
    x = ftr_ref[...]
    wp = wp_ref[...]
    fpc = jnp.maximum(
        jnp.dot(x, wp[0:8, 0:8], preferred_element_type=jnp.float32, precision=_PREC)
        + wp[8:9, 0:8], 0.0)
    row16 = jnp.concatenate(
        [xyz_ref[...], fpc, jnp.zeros((_FB, 5), jnp.float32)], axis=1)
    out_ref[...] = jnp.concatenate([row16] * 8, axis=1)


def _table1(ftr, xyzf, wpack):
    return pl.pallas_call(
        _table1_kernel,
        out_shape=jax.ShapeDtypeStruct((_P, 128), jnp.float32),
        grid=(_NFB,),
        in_specs=[
            pl.BlockSpec((_FB, 8), lambda j: (j, 0)),
            pl.BlockSpec((_FB, 3), lambda j: (j, 0)),
            pl.BlockSpec((16, 16), lambda j: (0, 0)),
        ],
        out_specs=pl.BlockSpec((_FB, 128), lambda j: (j, 0)),
    )(ftr, xyzf, wpack)


def _lanes_iota():
    return jax.lax.broadcasted_iota(jnp.int32, (1, 128), 1) % 16


def _dis_and_terms(g1, tab, s3_ref, d0_ref, wx_ref, wg_ref, brow_ref):
    xo = tab
    xo2 = jnp.repeat(xo.reshape(_PB, 1, 128), 2, axis=1).reshape(_RB, 128)
    rel = xo2 - g1
    dis2 = jnp.dot(rel * rel, s3_ref[...],
                   preferred_element_type=jnp.float32, precision=_PREC)
    disv = jnp.sqrt(dis2 + 1e-12)
    y = (jnp.dot(disv, d0_ref[...], preferred_element_type=jnp.float32, precision=_PREC)
         + jnp.dot(xo2, wx_ref[...], preferred_element_type=jnp.float32, precision=_PREC)
         + jnp.dot(g1, wg_ref[...], preferred_element_type=jnp.float32, precision=_PREC)
         + brow_ref[0:1, :])
    return y


def _mom2_kernel(g1_ref, tab_ref, s3_ref, d0_ref, wx_ref, wg_ref, brow_ref,
                 out_ref):
    g1 = g1_ref[...]
    y = _dis_and_terms(g1, tab_ref[...], s3_ref, d0_ref, wx_ref, wg_ref,
                       brow_ref)
    c = lax.dot_general(y, y, (((0,), (0,)), ((), ())),
                        preferred_element_type=jnp.float32, precision=_PREC)
    acc = jnp.zeros((16, 16), jnp.float32)
    for e in range(8):
        acc = acc + c[e * 16:(e + 1) * 16, e * 16:(e + 1) * 16]
    s = jnp.sum(y, axis=0, keepdims=True)
    sa = jnp.zeros((1, 16), jnp.float32)
    for e in range(8):
        sa = sa + s[:, e * 16:(e + 1) * 16]
    blk = jnp.concatenate(
        [acc, sa, jnp.zeros((7, 16), jnp.float32)], axis=0)

    @pl.when(pl.program_id(0) == 0)
    def _():
        out_ref[...] = jnp.zeros_like(out_ref)

    out_ref[...] += blk


def _mom2(g1v, table1, s3, d0, wx, wg, brow):
    return pl.pallas_call(
        _mom2_kernel,
        out_shape=jax.ShapeDtypeStruct((24, 16), jnp.float32),
        grid=(_NBLK,),
        in_specs=[
            pl.BlockSpec((_RB, 128), lambda i: (i, 0)),
            pl.BlockSpec((_PB, 128), lambda i: (i, 0)),
            pl.BlockSpec((128, 128), lambda i: (0, 0)),
            pl.BlockSpec((128, 128), lambda i: (0, 0)),
            pl.BlockSpec((128, 128), lambda i: (0, 0)),
            pl.BlockSpec((128, 128), lambda i: (0, 0)),
            pl.BlockSpec((8, 128), lambda i: (0, 0)),
        ],
        out_specs=pl.BlockSpec((24, 16), lambda i: (0, 0)),
    )(g1v, table1, s3, d0, wx, wg, brow)


def _pairfold(z):
    z3 = z.reshape(z.shape[0] // 2, 2, 128)
    return z3[:, 0, :] + z3[:, 1, :]


def _stage2_kernel(g1_ref, tab_ref, s3_ref, d0_ref, wx_ref, wg_ref, brow_ref,
                   wfc_ref, sg_ref, agg_ref, fx_ref, mom_ref):
    g1 = g1_ref[...]
    fxyz = jnp.maximum(
        _dis_and_terms(g1, tab_ref[...], s3_ref, d0_ref, wx_ref, wg_ref,
                       brow_ref), 0.0)
    lane = _lanes_iota()
    nbmask = jnp.logical_and(lane >= 3, lane < 11)
    fcat = jnp.where(nbmask, g1, 0.0) + fxyz
    att = jnp.dot(fcat, wfc_ref[...], preferred_element_type=jnp.float32, precision=_PREC)
    e = jnp.exp(att)
    stacked = jnp.concatenate([e, fcat * e], axis=0)
    sums = jnp.dot(stacked, sg_ref[...], preferred_element_type=jnp.float32, precision=_PREC)
    denom = _pairfold(sums[0:_RB])
    numer = _pairfold(sums[_RB:2 * _RB])
    agg = numer / denom
    agg_ref[...] = agg
    fx_ref[...] = fxyz

    ca = lax.dot_general(agg, agg, (((0,), (0,)), ((), ())),
                         preferred_element_type=jnp.float32, precision=_PREC)
    magg = ca[0:16, 0:16]
    sagg = jnp.sum(agg, axis=0, keepdims=True)[:, 0:16]
    cx = lax.dot_general(fxyz, fxyz, (((0,), (0,)), ((), ())),
                         preferred_element_type=jnp.float32, precision=_PREC)
    mfx = jnp.zeros((16, 16), jnp.float32)
    for eb in range(8):
        mfx = mfx + cx[eb * 16:(eb + 1) * 16, eb * 16:(eb + 1) * 16]
    sx = jnp.sum(fxyz, axis=0, keepdims=True)
    sfx = jnp.zeros((1, 16), jnp.float32)
    for eb in range(8):
        sfx = sfx + sx[:, eb * 16:(eb + 1) * 16]
    blk = jnp.concatenate(
        [magg, sagg, jnp.zeros((7, 16), jnp.float32),
         mfx, sfx, jnp.zeros((7, 16), jnp.float32)], axis=0)

    @pl.when(pl.program_id(0) == 0)
    def _():
        mom_ref[...] = jnp.zeros_like(mom_ref)

    mom_ref[...] += blk


def _stage2(g1v, table1, s3, d0, wx, wg, brow, wfc, sg):
    return pl.pallas_call(
        _stage2_kernel,
        out_shape=[
            jax.ShapeDtypeStruct((_P, 128), jnp.float32),
            jax.ShapeDtypeStruct((_M // 8, 128), jnp.float32),
            jax.ShapeDtypeStruct((48, 16), jnp.float32),
        ],
        grid=(_NBLK,),
        in_specs=[
            pl.BlockSpec((_RB, 128), lambda i: (i, 0)),
            pl.BlockSpec((_PB, 128), lambda i: (i, 0)),
            pl.BlockSpec((128, 128), lambda i: (0, 0)),
            pl.BlockSpec((128, 128), lambda i: (0, 0)),
            pl.BlockSpec((128, 128), lambda i: (0, 0)),
            pl.BlockSpec((128, 128), lambda i: (0, 0)),
            pl.BlockSpec((8, 128), lambda i: (0, 0)),
            pl.BlockSpec((128, 128), lambda i: (0, 0)),
            pl.BlockSpec((128, 128), lambda i: (0, 0)),
        ],
        out_specs=[
            pl.BlockSpec((_PB, 128), lambda i: (i, 0)),
            pl.BlockSpec((_RB, 128), lambda i: (i, 0)),
            pl.BlockSpec((48, 16), lambda i: (0, 0)),
        ],
    )(g1v, table1, s3, d0, wx, wg, brow, wfc, sg)


def _stage3_kernel(g2_ref, fx_ref, w3_ref, b3_ref, w4_ref, b4_ref,
                   wfc_ref, sg_ref, agg_ref, mom_ref):
    fnb2 = jnp.maximum(
        jnp.dot(g2_ref[...], w3_ref[...], preferred_element_type=jnp.float32, precision=_PREC)
        + b3_ref[0:1, :], 0.0)
    fx2 = jnp.maximum(
        jnp.dot(fx_ref[...], w4_ref[...], preferred_element_type=jnp.float32, precision=_PREC)
        + b4_ref[0:1, :], 0.0)
    fcat = fnb2 + fx2
    att = jnp.dot(fcat, wfc_ref[...], preferred_element_type=jnp.float32, precision=_PREC)
    e = jnp.exp(att)
    stacked = jnp.concatenate([e, fcat * e], axis=0)
    sums = jnp.dot(stacked, sg_ref[...], preferred_element_type=jnp.float32, precision=_PREC)
    denom = _pairfold(sums[0:_RB])
    numer = _pairfold(sums[_RB:2 * _RB])
    agg = numer / denom
    agg_ref[...] = agg

    ca = lax.dot_general(agg, agg, (((0,), (0,)), ((), ())),
                         preferred_element_type=jnp.float32, precision=_PREC)
    blk = jnp.concatenate(
        [ca[0:16, 0:16], jnp.sum(agg, axis=0, keepdims=True)[:, 0:16],
         jnp.zeros((7, 16), jnp.float32)], axis=0)

    @pl.when(pl.program_id(0) == 0)
    def _():
        mom_ref[...] = jnp.zeros_like(mom_ref)

    mom_ref[...] += blk


def _stage3(g2v, fxtab, w3, b3, w4, b4, wfc, sg):
    return pl.pallas_call(
        _stage3_kernel,
        out_shape=[
            jax.ShapeDtypeStruct((_P, 128), jnp.float32),
            jax.ShapeDtypeStruct((24, 16), jnp.float32),
        ],
        grid=(_NBLK,),
        in_specs=[
            pl.BlockSpec((_RB, 128), lambda i: (i, 0)),
            pl.BlockSpec((_RB, 128), lambda i: (i, 0)),
            pl.BlockSpec((128, 128), lambda i: (0, 0)),
            pl.BlockSpec((8, 128), lambda i: (0, 0)),
            pl.BlockSpec((128, 128), lambda i: (0, 0)),
            pl.BlockSpec((8, 128), lambda i: (0, 0)),
            pl.BlockSpec((128, 128), lambda i: (0, 0)),
            pl.BlockSpec((128, 128), lambda i: (0, 0)),
        ],
        out_specs=[
            pl.BlockSpec((_PB, 128), lambda i: (i, 0)),
            pl.BlockSpec((24, 16), lambda i: (0, 0)),
        ],
    )(g2v, fxtab, w3, b3, w4, b4, wfc, sg)


def _fpc2_kernel(agg_ref, wp_ref, out_ref, mom_ref):
    a = agg_ref[...][:, 0:16]
    wp = wp_ref[...]
    f = jnp.maximum(
        jnp.dot(a, wp[0:16, :], preferred_element_type=jnp.float32, precision=_PREC)
        + wp[16:17, :], 0.0)
    out_ref[...] = f
    aa = jnp.concatenate(
        [f, jnp.ones((_FB, 1), jnp.float32), jnp.zeros((_FB, 15), jnp.float32)],
        axis=1)
    m = lax.dot_general(aa, aa, (((0,), (0,)), ((), ())),
                        preferred_element_type=jnp.float32, precision=_PREC)

    @pl.when(pl.program_id(0) == 0)
    def _():
        mom_ref[...] = jnp.zeros_like(mom_ref)

    mom_ref[...] += m


def _fpc2(agg2tab, wpack):
    return pl.pallas_call(
        _fpc2_kernel,
        out_shape=[
            jax.ShapeDtypeStruct((_P, 16), jnp.float32),
            jax.ShapeDtypeStruct((32, 32), jnp.float32),
        ],
        grid=(_NFB,),
        in_specs=[
            pl.BlockSpec((_FB, 128), lambda j: (j, 0)),
            pl.BlockSpec((24, 16), lambda j: (0, 0)),
        ],
        out_specs=[
            pl.BlockSpec((_FB, 16), lambda j: (j, 0)),
            pl.BlockSpec((32, 32), lambda j: (0, 0)),
        ],
    )(agg2tab, wpack)


def _final_kernel(fpc2_ref, ftr_ref, w6_ref, wsc_ref, out_ref):
    w6 = w6_ref[...]
    wsc = wsc_ref[...]
    y = (jnp.dot(fpc2_ref[...], w6[0:16, :], preferred_element_type=jnp.float32, precision=_PREC)
         + w6[16:17, :]
         + jnp.dot(ftr_ref[...], wsc[0:8, :], preferred_element_type=jnp.float32, precision=_PREC)
         + wsc[8:9, :])
    out_ref[...] = jnp.where(y >= 0.0, y, 0.2 * y)


def _final(fpc2, ftr, w6pack, wscpack):
    return pl.pallas_call(
        _final_kernel,
        out_shape=jax.ShapeDtypeStruct((_P, 32), jnp.float32),
        grid=(_NFB,),
        in_specs=[
            pl.BlockSpec((_FB, 16), lambda j: (j, 0)),
            pl.BlockSpec((_FB, 8), lambda j: (j, 0)),
            pl.BlockSpec((24, 32), lambda j: (0, 0)),
            pl.BlockSpec((16, 32), lambda j: (0, 0)),
        ],
        out_specs=pl.BlockSpec((_FB, 32), lambda j: (j, 0)),
    )(fpc2, ftr, w6pack, wscpack)


def _np_zeros(*s):
    return np.zeros(s, np.float32)


def _build_s3():
    m = _np_zeros(128, 128)
    for e in range(8):
        for j in range(3):
            m[e * 16 + j, e * 16] = 1.0
    return jnp.asarray(m)


def _build_sg():
    m = _np_zeros(128, 128)
    for e in range(8):
        for ep in range(8):
            for s in range(16):
                m[e * 16 + s, ep * 16 + s] = 1.0
    return jnp.asarray(m)


def _build_relpos_consts(W2, b2, s2, t2, out_slots):
    W2s = W2 * s2[:, None]
    b2s = b2 * s2 + t2
    idx_out = np.asarray(out_slots)
    r3 = np.arange(3)
    d0b = jnp.zeros((16, 16), jnp.float32).at[0, idx_out].set(W2s[:, 0])
    wxb = jnp.zeros((16, 16), jnp.float32).at[np.ix_(r3, idx_out)].set(
        (W2s[:, 4:7] + W2s[:, 1:4]).T)
    wgb = jnp.zeros((16, 16), jnp.float32).at[np.ix_(r3, idx_out)].set(
        (W2s[:, 7:10] - W2s[:, 1:4]).T)
    bvec = jnp.zeros((16,), jnp.float32).at[idx_out].set(b2s)
    return (_bdiag_j(d0b), _bdiag_j(wxb), _bdiag_j(wgb), _rowpat(bvec))


def _fcat1_ch_of_slot():
    ch = [0] * 16
    for j in range(8):
        ch[3 + j] = j
    for j, s in enumerate(_XSLOT):
        ch[s] = 8 + j
    return ch


def kernel(feature, xyz, neigh_idx, params):
    p = params
    lfa = p['lfa']
    ftr = jnp.transpose(feature[..., 0], (0, 2, 1)).reshape(_P, 8)
    xyzf = xyz.reshape(_P, 3)

    momf = _mom_feat(ftr)
    s1f, m2f = momf[8, 0:8], momf[0:8, 0:8]
    W1f, b1f = _affine_from_in_moments(
        p['mlp1']['W'], p['mlp1']['b'], p['mlp1']['g'], p['mlp1']['be'],
        s1f, m2f, _P)
    Wscf, bscf = _affine_from_in_moments(
        p['shortcut']['W'], p['shortcut']['b'], p['shortcut']['g'],
        p['shortcut']['be'], s1f, m2f, _P)

    wpack1 = jnp.zeros((16, 16), jnp.float32)
    wpack1 = wpack1.at[0:8, 0:8].set(W1f.T).at[8, 0:8].set(b1f)
    table1 = _table1(ftr, xyzf, wpack1)

    g1v = _gatherv(table1, neigh_idx)

    s3 = _build_s3()
    sg = _build_sg()
    W2, b2 = lfa['mlp1']['W'], lfa['mlp1']['b']
    one8 = jnp.ones((8,), jnp.float32)
    zero8 = jnp.zeros((8,), jnp.float32)
    d0r, wxr, wgr, browr = _build_relpos_consts(
        W2, b2, one8, zero8, tuple(range(8)))
    mom2 = _mom2(g1v, table1, s3, d0r, wxr, wgr, browr)
    m2y, s2y = mom2[0:8, 0:8], mom2[16, 0:8]
    s2, t2 = _affine_from_out_moments(
        lfa['mlp1']['g'], lfa['mlp1']['be'], s2y, jnp.diag(m2y), _M)

    d0, wx, wg, brow = _build_relpos_consts(W2, b2, s2, t2, _XSLOT)
    ch1 = _fcat1_ch_of_slot()
    slot_of_ch1 = [0] * 16
    for s in range(16):
        slot_of_ch1[ch1[s]] = s
    sm1 = _slot_matrix_j(lfa['att1']['Wfc'], ch1, slot_of_ch1)
    wfc1 = _bdiag_j(sm1)
    aggtab, fxtab, mom34 = _stage2(
        g1v, table1, s3, d0, wx, wg, brow, wfc1, sg)
    magg, sagg = mom34[0:16, 0:16], mom34[16, 0:16]
    mfx, sfx = mom34[24:40, 0:16], mom34[40, 0:16]

    idx1 = jnp.asarray(slot_of_ch1)
    magg_c = magg[jnp.ix_(idx1, idx1)]
    sagg_c = sagg[idx1]
    W3f, b3f = _affine_from_in_moments(
        lfa['att1']['mlp']['W'], lfa['att1']['mlp']['b'],
        lfa['att1']['mlp']['g'], lfa['att1']['mlp']['be'],
        sagg_c, magg_c, _P)
    w3slot = _slot_matrix_j(W3f, ch1, list(range(8)))
    w3bd = _bdiag_j(w3slot)
    b3row = _rowpat(jnp.zeros((16,), jnp.float32).at[0:8].set(b3f))

    idxx = jnp.asarray(list(_XSLOT))
    mfx_c = mfx[jnp.ix_(idxx, idxx)]
    sfx_c = sfx[idxx]
    W4f, b4f = _affine_from_in_moments(
        lfa['mlp2']['W'], lfa['mlp2']['b'], lfa['mlp2']['g'],
        lfa['mlp2']['be'], sfx_c, mfx_c, _M)
    w4slot = _w4_slot_matrix(W4f)
    w4bd = _bdiag_j(w4slot)
    b4row = _rowpat(jnp.zeros((16,), jnp.float32).at[8:16].set(b4f))

    g2v = _gatherv(aggtab, neigh_idx)

    ch2 = list(range(16))
    sm2 = _slot_matrix_j(lfa['att2']['Wfc'], ch2, list(range(16)))
    wfc2 = _bdiag_j(sm2)
    agg2tab, mom5 = _stage3(
        g2v, fxtab, w3bd, b3row, w4bd, b4row, wfc2, sg)
    magg2, sagg2 = mom5[0:16, 0:16], mom5[16, 0:16]

    W5f, b5f = _affine_from_in_moments(
        lfa['att2']['mlp']['W'], lfa['att2']['mlp']['b'],
        lfa['att2']['mlp']['g'], lfa['att2']['mlp']['be'],
        sagg2, magg2, _P)
    wpack5 = jnp.zeros((24, 16), jnp.float32)
    wpack5 = wpack5.at[0:16, :].set(W5f.T).at[16, :].set(b5f)
    fpc2, mom6 = _fpc2(agg2tab, wpack5)
    m6, s6 = mom6[0:16, 0:16], mom6[16, 0:16]
    mu6 = s6 / _P
    cov6 = m6 / _P - jnp.outer(mu6, mu6)
    W6, b6 = p['mlp2']['W'], p['mlp2']['b']
    mean6 = W6 @ mu6 + b6
    var6 = jnp.einsum('oi,ij,oj->o', W6, cov6, W6)
    sca6 = p['mlp2']['g'] / jnp.sqrt(var6 + _EPS)
    W6f = W6 * sca6[:, None]
    b6f = sca6 * (b6 - mean6) + p['mlp2']['be']

    w6pack = jnp.zeros((24, 32), jnp.float32)
    w6pack = w6pack.at[0:16, :].set(W6f.T).at[16, :].set(b6f)
    wscpack = jnp.zeros((16, 32), jnp.float32)
    wscpack = wscpack.at[0:8, :].set(Wscf.T).at[8, :].set(bscf)

    out = _final(fpc2, ftr, w6pack, wscpack)
    out = out.reshape(_B, _N, 32)
    return jnp.transpose(out, (0, 2, 1))[..., None]


def _slot_matrix_j(Wfc, ch_of_slot, out_list):
    Wfc = jnp.asarray(Wfc)
    nout = Wfc.shape[0]
    m = jnp.zeros((16, 16), jnp.float32)
    for si in range(16):
        for o in range(nout):
            m = m.at[si, out_list[o]].set(Wfc[o, ch_of_slot[si]])
    return m


def _w4_slot_matrix(W4f):
    m = jnp.zeros((16, 16), jnp.float32)
    for j in range(8):
        for o in range(8):
            m = m.at[_XSLOT[j], 8 + o].set(W4f[o, j])
    return m


# speedup vs baseline: 1.2163x; 1.2163x over previous
import functools

import jax
import jax.numpy as jnp
import numpy as np
from jax import lax
from jax.experimental import pallas as pl
from jax.experimental.pallas import tpu as pltpu
from jax.experimental.pallas import tpu_sc as plsc

_B, _N, _K = 4, 50000, 16
_NK = _N * _K
_M = _B * _NK
_P = _B * _N
_CH = 128
_NCHUNK = _M // _CH
_CPB = _NK // _CH
_NW = 32
_ROUNDS = -(-_NCHUNK // _NW)

_RB = 640
_PB = _RB // 2
_NBLK = (_M // 8) // _RB
_FB = 1600
_NFB = _P // _FB

_XSLOT = (11, 12, 13, 14, 15, 0, 1, 2)
_EPS = 1e-5
_PREC = jax.lax.Precision.HIGHEST


def _sc_gather_body(table_hbm, idx_hbm, out_hbm, idx_v, rows_v, sem):
    wid = lax.axis_index("s") * 2 + lax.axis_index("c")

    def body(r, _):
        c = wid + r * _NW

        @pl.when(c < _NCHUNK)
        def _():
            pltpu.sync_copy(idx_hbm.at[pl.ds(c * _CH, _CH)], idx_v)
            b_off = (c // _CPB) * _N
            for t in range(_CH // 16):
                sl = pl.ds(t * 16, 16)
                idx_v[sl] = (idx_v[sl] + b_off) * 8
            pltpu.async_copy(table_hbm.at[idx_v], rows_v, sem).wait()
            pltpu.sync_copy(rows_v, out_hbm.at[pl.ds(c * _CH, _CH)])

        return ()

    lax.fori_loop(0, _ROUNDS, body, ())


@functools.lru_cache(maxsize=1)
def _get_sc_gather():
    @functools.partial(
        pl.kernel,
        out_type=jax.ShapeDtypeStruct((_M, 16), jnp.float32),
        mesh=plsc.VectorSubcoreMesh(core_axis_name="c", subcore_axis_name="s"),
        compiler_params=pltpu.CompilerParams(use_tc_tiling_on_sc=False),
        scratch_types=[
            pltpu.VMEM((_CH,), jnp.int32),
            pltpu.VMEM((_CH, 16), jnp.float32),
            pltpu.SemaphoreType.DMA,
        ],
    )
    def _sc_gather(table_hbm, idx_hbm, out_hbm, idx_v, rows_v, sem):
        _sc_gather_body(table_hbm, idx_hbm, out_hbm, idx_v, rows_v, sem)

    return _sc_gather


def _gatherv(table128, neigh_idx):
    out = _get_sc_gather()(table128.reshape(8 * _P, 16), neigh_idx.reshape(_M))
    return out.reshape(_M // 8, 128)


def _affine_from_in_moments(W, b, g, be, s1, m2, cnt):
    mu = s1 / cnt
    cov = m2 / cnt - jnp.outer(mu, mu)
    mean = W @ mu + b
    var = jnp.einsum('oi,ij,oj->o', W, cov, W)
    s = g / jnp.sqrt(var + _EPS)
    return W * s[:, None], s * (b - mean) + be


def _affine_from_out_moments(g, be, s1, m2d, cnt):
    mean = s1 / cnt
    var = m2d / cnt - mean * mean
    s = g / jnp.sqrt(var + _EPS)
    return s, be - mean * s


def _bdiag_j(mat16):
    e = jnp.zeros((128, 128), jnp.float32)
    for i in range(8):
        e = e.at[i * 16:(i + 1) * 16, i * 16:(i + 1) * 16].set(mat16)
    return e


def _rowpat(vec16):
    return jnp.tile(vec16, 8)[None, :].repeat(8, 0)


def _mom_feat_kernel(ftr_ref, out_ref):
    x = ftr_ref[...]
    a = jnp.concatenate(
        [x, jnp.ones((_FB, 1), jnp.float32), jnp.zeros((_FB, 7), jnp.float32)],
        axis=1)
    m = lax.dot_general(a, a, (((0,), (0,)), ((), ())),
                        preferred_element_type=jnp.float32, precision=_PREC)

    @pl.when(pl.program_id(0) == 0)
    def _():
        out_ref[...] = jnp.zeros_like(out_ref)

    out_ref[...] += m


def _mom_feat(ftr):
    return pl.pallas_call(
        _mom_feat_kernel,
        out_shape=jax.ShapeDtypeStruct((16, 16), jnp.float32),
        grid=(_NFB,),
        in_specs=[pl.BlockSpec((_FB, 8), lambda j: (j, 0))],
        out_specs=pl.BlockSpec((16, 16), lambda j: (0, 0)),
    )(ftr)


def _table1_kernel(ftr_ref, xyz_ref, wp_ref, out_ref):
    x = ftr_ref[...]
    wp = wp_ref[...]
    fpc = jnp.maximum(
        jnp.dot(x, wp[0:8, 0:8], preferred_element_type=jnp.float32, precision=_PREC)
        + wp[8:9, 0:8], 0.0)
    row16 = jnp.concatenate(
        [xyz_ref[...], fpc, jnp.zeros((_FB, 5), jnp.float32)], axis=1)
    out_ref[...] = jnp.concatenate([row16] * 8, axis=1)


def _table1(ftr, xyzf, wpack):
    return pl.pallas_call(
        _table1_kernel,
        out_shape=jax.ShapeDtypeStruct((_P, 128), jnp.float32),
        grid=(_NFB,),
        in_specs=[
            pl.BlockSpec((_FB, 8), lambda j: (j, 0)),
            pl.BlockSpec((_FB, 3), lambda j: (j, 0)),
            pl.BlockSpec((16, 16), lambda j: (0, 0)),
        ],
        out_specs=pl.BlockSpec((_FB, 128), lambda j: (j, 0)),
    )(ftr, xyzf, wpack)


def _lanes_iota():
    return jax.lax.broadcasted_iota(jnp.int32, (1, 128), 1) % 16


def _dis_and_terms(g1, tab, s3_ref, d0_ref, wx_ref, wg_ref, brow_ref):
    xo = tab
    xo2 = jnp.repeat(xo.reshape(_PB, 1, 128), 2, axis=1).reshape(_RB, 128)
    rel = xo2 - g1
    dis2 = jnp.dot(rel * rel, s3_ref[...],
                   preferred_element_type=jnp.float32, precision=_PREC)
    disv = jnp.sqrt(dis2 + 1e-12)
    y = (jnp.dot(disv, d0_ref[...], preferred_element_type=jnp.float32, precision=_PREC)
         + jnp.dot(xo2, wx_ref[...], preferred_element_type=jnp.float32, precision=_PREC)
         + jnp.dot(g1, wg_ref[...], preferred_element_type=jnp.float32, precision=_PREC)
         + brow_ref[0:1, :])
    return y


def _mom2_kernel(g1_ref, tab_ref, s3_ref, d0_ref, wx_ref, wg_ref, brow_ref,
                 out_ref):
    g1 = g1_ref[...]
    y = _dis_and_terms(g1, tab_ref[...], s3_ref, d0_ref, wx_ref, wg_ref,
                       brow_ref)
    c = lax.dot_general(y, y, (((0,), (0,)), ((), ())),
                        preferred_element_type=jnp.float32, precision=_PREC)
    acc = jnp.zeros((16, 16), jnp.float32)
    for e in range(8):
        acc = acc + c[e * 16:(e + 1) * 16, e * 16:(e + 1) * 16]
    s = jnp.sum(y, axis=0, keepdims=True)
    sa = jnp.zeros((1, 16), jnp.float32)
    for e in range(8):
        sa = sa + s[:, e * 16:(e + 1) * 16]
    blk = jnp.concatenate(
        [acc, sa, jnp.zeros((7, 16), jnp.float32)], axis=0)

    @pl.when(pl.program_id(0) == 0)
    def _():
        out_ref[...] = jnp.zeros_like(out_ref)

    out_ref[...] += blk


def _mom2(g1v, table1, s3, d0, wx, wg, brow):
    return pl.pallas_call(
        _mom2_kernel,
        out_shape=jax.ShapeDtypeStruct((24, 16), jnp.float32),
        grid=(_NBLK,),
        in_specs=[
            pl.BlockSpec((_RB, 128), lambda i: (i, 0)),
            pl.BlockSpec((_PB, 128), lambda i: (i, 0)),
            pl.BlockSpec((128, 128), lambda i: (0, 0)),
            pl.BlockSpec((128, 128), lambda i: (0, 0)),
            pl.BlockSpec((128, 128), lambda i: (0, 0)),
            pl.BlockSpec((128, 128), lambda i: (0, 0)),
            pl.BlockSpec((8, 128), lambda i: (0, 0)),
        ],
        out_specs=pl.BlockSpec((24, 16), lambda i: (0, 0)),
    )(g1v, table1, s3, d0, wx, wg, brow)


def _pairfold(z):
    z3 = z.reshape(z.shape[0] // 2, 2, 128)
    return z3[:, 0, :] + z3[:, 1, :]


def _groupsum(y):
    y = y + jnp.roll(y, 16, axis=1)
    y = y + jnp.roll(y, 32, axis=1)
    y = y + jnp.roll(y, 64, axis=1)
    return y


def _stage2_kernel(g1_ref, tab_ref, s3_ref, d0_ref, wx_ref, wg_ref, brow_ref,
                   wfc_ref, sg_ref, agg_ref, fx_ref, mom_ref):
    g1 = g1_ref[...]
    fxyz = jnp.maximum(
        _dis_and_terms(g1, tab_ref[...], s3_ref, d0_ref, wx_ref, wg_ref,
                       brow_ref), 0.0)
    lane = _lanes_iota()
    nbmask = jnp.logical_and(lane >= 3, lane < 11)
    fcat = jnp.where(nbmask, g1, 0.0) + fxyz
    att = jnp.dot(fcat, wfc_ref[...], preferred_element_type=jnp.float32, precision=_PREC)
    e = jnp.exp(att)
    denom = _pairfold(_groupsum(e))
    numer = _pairfold(_groupsum(fcat * e))
    agg = numer / denom
    agg_ref[...] = agg
    fx_ref[...] = fxyz

    ca = lax.dot_general(agg, agg, (((0,), (0,)), ((), ())),
                         preferred_element_type=jnp.float32, precision=_PREC)
    magg = ca[0:16, 0:16]
    sagg = jnp.sum(agg, axis=0, keepdims=True)[:, 0:16]
    cx = lax.dot_general(fxyz, fxyz, (((0,), (0,)), ((), ())),
                         preferred_element_type=jnp.float32, precision=_PREC)
    mfx = jnp.zeros((16, 16), jnp.float32)
    for eb in range(8):
        mfx = mfx + cx[eb * 16:(eb + 1) * 16, eb * 16:(eb + 1) * 16]
    sx = jnp.sum(fxyz, axis=0, keepdims=True)
    sfx = jnp.zeros((1, 16), jnp.float32)
    for eb in range(8):
        sfx = sfx + sx[:, eb * 16:(eb + 1) * 16]
    blk = jnp.concatenate(
        [magg, sagg, jnp.zeros((7, 16), jnp.float32),
         mfx, sfx, jnp.zeros((7, 16), jnp.float32)], axis=0)

    @pl.when(pl.program_id(0) == 0)
    def _():
        mom_ref[...] = jnp.zeros_like(mom_ref)

    mom_ref[...] += blk


def _stage2(g1v, table1, s3, d0, wx, wg, brow, wfc, sg):
    return pl.pallas_call(
        _stage2_kernel,
        out_shape=[
            jax.ShapeDtypeStruct((_P, 128), jnp.float32),
            jax.ShapeDtypeStruct((_M // 8, 128), jnp.float32),
            jax.ShapeDtypeStruct((48, 16), jnp.float32),
        ],
        grid=(_NBLK,),
        in_specs=[
            pl.BlockSpec((_RB, 128), lambda i: (i, 0)),
            pl.BlockSpec((_PB, 128), lambda i: (i, 0)),
            pl.BlockSpec((128, 128), lambda i: (0, 0)),
            pl.BlockSpec((128, 128), lambda i: (0, 0)),
            pl.BlockSpec((128, 128), lambda i: (0, 0)),
            pl.BlockSpec((128, 128), lambda i: (0, 0)),
            pl.BlockSpec((8, 128), lambda i: (0, 0)),
            pl.BlockSpec((128, 128), lambda i: (0, 0)),
            pl.BlockSpec((128, 128), lambda i: (0, 0)),
        ],
        out_specs=[
            pl.BlockSpec((_PB, 128), lambda i: (i, 0)),
            pl.BlockSpec((_RB, 128), lambda i: (i, 0)),
            pl.BlockSpec((48, 16), lambda i: (0, 0)),
        ],
    )(g1v, table1, s3, d0, wx, wg, brow, wfc, sg)


def _stage3_kernel(g2_ref, fx_ref, w3_ref, b3_ref, w4_ref, b4_ref,
                   wfc_ref, sg_ref, agg_ref, mom_ref):
    fnb2 = jnp.maximum(
        jnp.dot(g2_ref[...], w3_ref[...], preferred_element_type=jnp.float32, precision=_PREC)
        + b3_ref[0:1, :], 0.0)
    fx2 = jnp.maximum(
        jnp.dot(fx_ref[...], w4_ref[...], preferred_element_type=jnp.float32, precision=_PREC)
        + b4_ref[0:1, :], 0.0)
    fcat = fnb2 + fx2
    att = jnp.dot(fcat, wfc_ref[...], preferred_element_type=jnp.float32, precision=_PREC)
    e = jnp.exp(att)
    denom = _pairfold(_groupsum(e))
    numer = _pairfold(_groupsum(fcat * e))
    agg = numer / denom
    agg_ref[...] = agg

    ca = lax.dot_general(agg, agg, (((0,), (0,)), ((), ())),
                         preferred_element_type=jnp.float32, precision=_PREC)
    blk = jnp.concatenate(
        [ca[0:16, 0:16], jnp.sum(agg, axis=0, keepdims=True)[:, 0:16],
         jnp.zeros((7, 16), jnp.float32)], axis=0)

    @pl.when(pl.program_id(0) == 0)
    def _():
        mom_ref[...] = jnp.zeros_like(mom_ref)

    mom_ref[...] += blk


def _stage3(g2v, fxtab, w3, b3, w4, b4, wfc, sg):
    return pl.pallas_call(
        _stage3_kernel,
        out_shape=[
            jax.ShapeDtypeStruct((_P, 128), jnp.float32),
            jax.ShapeDtypeStruct((24, 16), jnp.float32),
        ],
        grid=(_NBLK,),
        in_specs=[
            pl.BlockSpec((_RB, 128), lambda i: (i, 0)),
            pl.BlockSpec((_RB, 128), lambda i: (i, 0)),
            pl.BlockSpec((128, 128), lambda i: (0, 0)),
            pl.BlockSpec((8, 128), lambda i: (0, 0)),
            pl.BlockSpec((128, 128), lambda i: (0, 0)),
            pl.BlockSpec((8, 128), lambda i: (0, 0)),
            pl.BlockSpec((128, 128), lambda i: (0, 0)),
            pl.BlockSpec((128, 128), lambda i: (0, 0)),
        ],
        out_specs=[
            pl.BlockSpec((_PB, 128), lambda i: (i, 0)),
            pl.BlockSpec((24, 16), lambda i: (0, 0)),
        ],
    )(g2v, fxtab, w3, b3, w4, b4, wfc, sg)


def _fpc2_kernel(agg_ref, wp_ref, out_ref, mom_ref):
    a = agg_ref[...][:, 0:16]
    wp = wp_ref[...]
    f = jnp.maximum(
        jnp.dot(a, wp[0:16, :], preferred_element_type=jnp.float32, precision=_PREC)
        + wp[16:17, :], 0.0)
    out_ref[...] = f
    aa = jnp.concatenate(
        [f, jnp.ones((_FB, 1), jnp.float32), jnp.zeros((_FB, 15), jnp.float32)],
        axis=1)
    m = lax.dot_general(aa, aa, (((0,), (0,)), ((), ())),
                        preferred_element_type=jnp.float32, precision=_PREC)

    @pl.when(pl.program_id(0) == 0)
    def _():
        mom_ref[...] = jnp.zeros_like(mom_ref)

    mom_ref[...] += m


def _fpc2(agg2tab, wpack):
    return pl.pallas_call(
        _fpc2_kernel,
        out_shape=[
            jax.ShapeDtypeStruct((_P, 16), jnp.float32),
            jax.ShapeDtypeStruct((32, 32), jnp.float32),
        ],
        grid=(_NFB,),
        in_specs=[
            pl.BlockSpec((_FB, 128), lambda j: (j, 0)),
            pl.BlockSpec((24, 16), lambda j: (0, 0)),
        ],
        out_specs=[
            pl.BlockSpec((_FB, 16), lambda j: (j, 0)),
            pl.BlockSpec((32, 32), lambda j: (0, 0)),
        ],
    )(agg2tab, wpack)


def _final_kernel(fpc2_ref, ftr_ref, w6_ref, wsc_ref, out_ref):
    w6 = w6_ref[...]
    wsc = wsc_ref[...]
    y = (jnp.dot(fpc2_ref[...], w6[0:16, :], preferred_element_type=jnp.float32, precision=_PREC)
         + w6[16:17, :]
         + jnp.dot(ftr_ref[...], wsc[0:8, :], preferred_element_type=jnp.float32, precision=_PREC)
         + wsc[8:9, :])
    out_ref[...] = jnp.where(y >= 0.0, y, 0.2 * y)


def _final(fpc2, ftr, w6pack, wscpack):
    return pl.pallas_call(
        _final_kernel,
        out_shape=jax.ShapeDtypeStruct((_P, 32), jnp.float32),
        grid=(_NFB,),
        in_specs=[
            pl.BlockSpec((_FB, 16), lambda j: (j, 0)),
            pl.BlockSpec((_FB, 8), lambda j: (j, 0)),
            pl.BlockSpec((24, 32), lambda j: (0, 0)),
            pl.BlockSpec((16, 32), lambda j: (0, 0)),
        ],
        out_specs=pl.BlockSpec((_FB, 32), lambda j: (j, 0)),
    )(fpc2, ftr, w6pack, wscpack)


def _np_zeros(*s):
    return np.zeros(s, np.float32)


def _build_s3():
    m = _np_zeros(128, 128)
    for e in range(8):
        for j in range(3):
            m[e * 16 + j, e * 16] = 1.0
    return jnp.asarray(m)


def _build_sg():
    m = _np_zeros(128, 128)
    for e in range(8):
        for ep in range(8):
            for s in range(16):
                m[e * 16 + s, ep * 16 + s] = 1.0
    return jnp.asarray(m)


def _build_relpos_consts(W2, b2, s2, t2, out_slots):
    W2s = W2 * s2[:, None]
    b2s = b2 * s2 + t2
    idx_out = np.asarray(out_slots)
    r3 = np.arange(3)
    d0b = jnp.zeros((16, 16), jnp.float32).at[0, idx_out].set(W2s[:, 0])
    wxb = jnp.zeros((16, 16), jnp.float32).at[np.ix_(r3, idx_out)].set(
        (W2s[:, 4:7] + W2s[:, 1:4]).T)
    wgb = jnp.zeros((16, 16), jnp.float32).at[np.ix_(r3, idx_out)].set(
        (W2s[:, 7:10] - W2s[:, 1:4]).T)
    bvec = jnp.zeros((16,), jnp.float32).at[idx_out].set(b2s)
    return (_bdiag_j(d0b), _bdiag_j(wxb), _bdiag_j(wgb), _rowpat(bvec))


def _fcat1_ch_of_slot():
    ch = [0] * 16
    for j in range(8):
        ch[3 + j] = j
    for j, s in enumerate(_XSLOT):
        ch[s] = 8 + j
    return ch


def kernel(feature, xyz, neigh_idx, params):
    p = params
    lfa = p['lfa']
    ftr = jnp.transpose(feature[..., 0], (0, 2, 1)).reshape(_P, 8)
    xyzf = xyz.reshape(_P, 3)

    momf = _mom_feat(ftr)
    s1f, m2f = momf[8, 0:8], momf[0:8, 0:8]
    W1f, b1f = _affine_from_in_moments(
        p['mlp1']['W'], p['mlp1']['b'], p['mlp1']['g'], p['mlp1']['be'],
        s1f, m2f, _P)
    Wscf, bscf = _affine_from_in_moments(
        p['shortcut']['W'], p['shortcut']['b'], p['shortcut']['g'],
        p['shortcut']['be'], s1f, m2f, _P)

    wpack1 = jnp.zeros((16, 16), jnp.float32)
    wpack1 = wpack1.at[0:8, 0:8].set(W1f.T).at[8, 0:8].set(b1f)
    table1 = _table1(ftr, xyzf, wpack1)

    g1v = _gatherv(table1, neigh_idx)

    s3 = _build_s3()
    sg = _build_sg()
    W2, b2 = lfa['mlp1']['W'], lfa['mlp1']['b']
    one8 = jnp.ones((8,), jnp.float32)
    zero8 = jnp.zeros((8,), jnp.float32)
    d0r, wxr, wgr, browr = _build_relpos_consts(
        W2, b2, one8, zero8, tuple(range(8)))
    mom2 = _mom2(g1v, table1, s3, d0r, wxr, wgr, browr)
    m2y, s2y = mom2[0:8, 0:8], mom2[16, 0:8]
    s2, t2 = _affine_from_out_moments(
        lfa['mlp1']['g'], lfa['mlp1']['be'], s2y, jnp.diag(m2y), _M)

    d0, wx, wg, brow = _build_relpos_consts(W2, b2, s2, t2, _XSLOT)
    ch1 = _fcat1_ch_of_slot()
    slot_of_ch1 = [0] * 16
    for s in range(16):
        slot_of_ch1[ch1[s]] = s
    sm1 = _slot_matrix_j(lfa['att1']['Wfc'], ch1, slot_of_ch1)
    wfc1 = _bdiag_j(sm1)
    aggtab, fxtab, mom34 = _stage2(
        g1v, table1, s3, d0, wx, wg, brow, wfc1, sg)
    magg, sagg = mom34[0:16, 0:16], mom34[16, 0:16]
    mfx, sfx = mom34[24:40, 0:16], mom34[40, 0:16]

    idx1 = jnp.asarray(slot_of_ch1)
    magg_c = magg[jnp.ix_(idx1, idx1)]
    sagg_c = sagg[idx1]
    W3f, b3f = _affine_from_in_moments(
        lfa['att1']['mlp']['W'], lfa['att1']['mlp']['b'],
        lfa['att1']['mlp']['g'], lfa['att1']['mlp']['be'],
        sagg_c, magg_c, _P)
    w3slot = _slot_matrix_j(W3f, ch1, list(range(8)))
    w3bd = _bdiag_j(w3slot)
    b3row = _rowpat(jnp.zeros((16,), jnp.float32).at[0:8].set(b3f))

    idxx = jnp.asarray(list(_XSLOT))
    mfx_c = mfx[jnp.ix_(idxx, idxx)]
    sfx_c = sfx[idxx]
    W4f, b4f = _affine_from_in_moments(
        lfa['mlp2']['W'], lfa['mlp2']['b'], lfa['mlp2']['g'],
        lfa['mlp2']['be'], sfx_c, mfx_c, _M)
    w4slot = _w4_slot_matrix(W4f)
    w4bd = _bdiag_j(w4slot)
    b4row = _rowpat(jnp.zeros((16,), jnp.float32).at[8:16].set(b4f))

    g2v = _gatherv(aggtab, neigh_idx)

    ch2 = list(range(16))
    sm2 = _slot_matrix_j(lfa['att2']['Wfc'], ch2, list(range(16)))
    wfc2 = _bdiag_j(sm2)
    agg2tab, mom5 = _stage3(
        g2v, fxtab, w3bd, b3row, w4bd, b4row, wfc2, sg)
    magg2, sagg2 = mom5[0:16, 0:16], mom5[16, 0:16]

    W5f, b5f = _affine_from_in_moments(
        lfa['att2']['mlp']['W'], lfa['att2']['mlp']['b'],
        lfa['att2']['mlp']['g'], lfa['att2']['mlp']['be'],
        sagg2, magg2, _P)
    wpack5 = jnp.zeros((24, 16), jnp.float32)
    wpack5 = wpack5.at[0:16, :].set(W5f.T).at[16, :].set(b5f)
    fpc2, mom6 = _fpc2(agg2tab, wpack5)
    m6, s6 = mom6[0:16, 0:16], mom6[16, 0:16]
    mu6 = s6 / _P
    cov6 = m6 / _P - jnp.outer(mu6, mu6)
    W6, b6 = p['mlp2']['W'], p['mlp2']['b']
    mean6 = W6 @ mu6 + b6
    var6 = jnp.einsum('oi,ij,oj->o', W6, cov6, W6)
    sca6 = p['mlp2']['g'] / jnp.sqrt(var6 + _EPS)
    W6f = W6 * sca6[:, None]
    b6f = sca6 * (b6 - mean6) + p['mlp2']['be']

    w6pack = jnp.zeros((24, 32), jnp.float32)
    w6pack = w6pack.at[0:16, :].set(W6f.T).at[16, :].set(b6f)
    wscpack = jnp.zeros((16, 32), jnp.float32)
    wscpack = wscpack.at[0:8, :].set(Wscf.T).at[8, :].set(bscf)

    out = _final(fpc2, ftr, w6pack, wscpack)
    out = out.reshape(_B, _N, 32)
    return jnp.transpose(out, (0, 2, 1))[..., None]


def _slot_matrix_j(Wfc, ch_of_slot, out_list):
    Wfc = jnp.asarray(Wfc)
    nout = Wfc.shape[0]
    m = jnp.zeros((16, 16), jnp.float32)
    for si in range(16):
        for o in range(nout):
            m = m.at[si, out_list[o]].set(Wfc[o, ch_of_slot[si]])
    return m


def _w4_slot_matrix(W4f):
    m = jnp.zeros((16, 16), jnp.float32)
    for j in range(8):
        for o in range(8):
            m = m.at[_XSLOT[j], 8 + o].set(W4f[o, j])
    return m


# speedup vs baseline: 1.3382x; 1.1002x over previous
import functools

import jax
import jax.numpy as jnp
import numpy as np
from jax import lax
from jax.experimental import pallas as pl
from jax.experimental.pallas import tpu as pltpu
from jax.experimental.pallas import tpu_sc as plsc

_B, _N, _K = 4, 50000, 16
_NK = _N * _K
_M = _B * _NK
_P = _B * _N
_CH = 128
_NCHUNK = _M // _CH
_CPB = _NK // _CH
_NW = 32
_ROUNDS = -(-_NCHUNK // _NW)

_RB = 640
_PB = _RB // 2
_NBLK = (_M // 8) // _RB
_FB = 1600
_NFB = _P // _FB

_XSLOT = (11, 12, 13, 14, 15, 0, 1, 2)
_EPS = 1e-5
_PREC = jax.lax.Precision.HIGHEST


def _sc_gather_body(table_hbm, idx_hbm, out_hbm, idx_v, rows_v, sem):
    wid = lax.axis_index("s") * 2 + lax.axis_index("c")

    def body(r, _):
        c = wid + r * _NW

        @pl.when(c < _NCHUNK)
        def _():
            pltpu.sync_copy(idx_hbm.at[pl.ds(c * _CH, _CH)], idx_v)
            b_off = (c // _CPB) * _N
            for t in range(_CH // 16):
                sl = pl.ds(t * 16, 16)
                idx_v[sl] = (idx_v[sl] + b_off) * 8
            pltpu.async_copy(table_hbm.at[idx_v], rows_v, sem).wait()
            pltpu.sync_copy(rows_v, out_hbm.at[pl.ds(c * _CH, _CH)])

        return ()

    lax.fori_loop(0, _ROUNDS, body, ())


@functools.lru_cache(maxsize=1)
def _get_sc_gather():
    @functools.partial(
        pl.kernel,
        out_type=jax.ShapeDtypeStruct((_M, 16), jnp.float32),
        mesh=plsc.VectorSubcoreMesh(core_axis_name="c", subcore_axis_name="s"),
        compiler_params=pltpu.CompilerParams(use_tc_tiling_on_sc=False),
        scratch_types=[
            pltpu.VMEM((_CH,), jnp.int32),
            pltpu.VMEM((_CH, 16), jnp.float32),
            pltpu.SemaphoreType.DMA,
        ],
    )
    def _sc_gather(table_hbm, idx_hbm, out_hbm, idx_v, rows_v, sem):
        _sc_gather_body(table_hbm, idx_hbm, out_hbm, idx_v, rows_v, sem)

    return _sc_gather


def _gatherv(table128, neigh_idx):
    out = _get_sc_gather()(table128.reshape(8 * _P, 16), neigh_idx.reshape(_M))
    return out.reshape(_M // 8, 128)


def _affine_from_in_moments(W, b, g, be, s1, m2, cnt):
    mu = s1 / cnt
    cov = m2 / cnt - jnp.outer(mu, mu)
    mean = W @ mu + b
    var = jnp.einsum('oi,ij,oj->o', W, cov, W)
    s = g / jnp.sqrt(var + _EPS)
    return W * s[:, None], s * (b - mean) + be


def _affine_from_out_moments(g, be, s1, m2d, cnt):
    mean = s1 / cnt
    var = m2d / cnt - mean * mean
    s = g / jnp.sqrt(var + _EPS)
    return s, be - mean * s


def _bdiag_j(mat16):
    e = jnp.zeros((128, 128), jnp.float32)
    for i in range(8):
        e = e.at[i * 16:(i + 1) * 16, i * 16:(i + 1) * 16].set(mat16)
    return e


def _rowpat(vec16):
    return jnp.tile(vec16, 8)[None, :].repeat(8, 0)


def _mom_feat_kernel(ftr_ref, out_ref):
    x = ftr_ref[...]
    a = jnp.concatenate(
        [x, jnp.ones((_FB, 1), jnp.float32), jnp.zeros((_FB, 7), jnp.float32)],
        axis=1)
    m = lax.dot_general(a, a, (((0,), (0,)), ((), ())),
                        preferred_element_type=jnp.float32)

    @pl.when(pl.program_id(0) == 0)
    def _():
        out_ref[...] = jnp.zeros_like(out_ref)

    out_ref[...] += m


def _mom_feat(ftr):
    return pl.pallas_call(
        _mom_feat_kernel,
        out_shape=jax.ShapeDtypeStruct((16, 16), jnp.float32),
        grid=(_NFB,),
        in_specs=[pl.BlockSpec((_FB, 8), lambda j: (j, 0))],
        out_specs=pl.BlockSpec((16, 16), lambda j: (0, 0)),
    )(ftr)


def _table1_kernel(ftr_ref, xyz_ref, wp_ref, out_ref):
    x = ftr_ref[...]
    wp = wp_ref[...]
    fpc = jnp.maximum(
        jnp.dot(x, wp[0:8, 0:8], preferred_element_type=jnp.float32, precision=_PREC)
        + wp[8:9, 0:8], 0.0)
    row16 = jnp.concatenate(
        [xyz_ref[...], fpc, jnp.zeros((_FB, 5), jnp.float32)], axis=1)
    out_ref[...] = jnp.concatenate([row16] * 8, axis=1)


def _table1(ftr, xyzf, wpack):
    return pl.pallas_call(
        _table1_kernel,
        out_shape=jax.ShapeDtypeStruct((_P, 128), jnp.float32),
        grid=(_NFB,),
        in_specs=[
            pl.BlockSpec((_FB, 8), lambda j: (j, 0)),
            pl.BlockSpec((_FB, 3), lambda j: (j, 0)),
            pl.BlockSpec((16, 16), lambda j: (0, 0)),
        ],
        out_specs=pl.BlockSpec((_FB, 128), lambda j: (j, 0)),
    )(ftr, xyzf, wpack)


def _lanes_iota():
    return jax.lax.broadcasted_iota(jnp.int32, (1, 128), 1) % 16


def _dis_and_terms(g1, tab, s3_ref, d0_ref, wx_ref, wg_ref, brow_ref):
    xo = tab
    xo2 = jnp.repeat(xo.reshape(_PB, 1, 128), 2, axis=1).reshape(_RB, 128)
    rel = xo2 - g1
    dis2 = jnp.dot(rel * rel, s3_ref[...],
                   preferred_element_type=jnp.float32, precision=_PREC)
    disv = jnp.sqrt(dis2 + 1e-12)
    y = (jnp.dot(disv, d0_ref[...], preferred_element_type=jnp.float32, precision=_PREC)
         + jnp.dot(xo2, wx_ref[...], preferred_element_type=jnp.float32, precision=_PREC)
         + jnp.dot(g1, wg_ref[...], preferred_element_type=jnp.float32, precision=_PREC)
         + brow_ref[0:1, :])
    return y


def _mom2_kernel(g1_ref, tab_ref, s3_ref, d0_ref, wx_ref, wg_ref, brow_ref,
                 out_ref):
    g1 = g1_ref[...]
    y = _dis_and_terms(g1, tab_ref[...], s3_ref, d0_ref, wx_ref, wg_ref,
                       brow_ref)
    c = lax.dot_general(y, y, (((0,), (0,)), ((), ())),
                        preferred_element_type=jnp.float32)
    acc = jnp.zeros((16, 16), jnp.float32)
    for e in range(8):
        acc = acc + c[e * 16:(e + 1) * 16, e * 16:(e + 1) * 16]
    s = jnp.sum(y, axis=0, keepdims=True)
    sa = jnp.zeros((1, 16), jnp.float32)
    for e in range(8):
        sa = sa + s[:, e * 16:(e + 1) * 16]
    blk = jnp.concatenate(
        [acc, sa, jnp.zeros((7, 16), jnp.float32)], axis=0)

    @pl.when(pl.program_id(0) == 0)
    def _():
        out_ref[...] = jnp.zeros_like(out_ref)

    out_ref[...] += blk


def _mom2(g1v, table1, s3, d0, wx, wg, brow):
    return pl.pallas_call(
        _mom2_kernel,
        out_shape=jax.ShapeDtypeStruct((24, 16), jnp.float32),
        grid=(_NBLK,),
        in_specs=[
            pl.BlockSpec((_RB, 128), lambda i: (i, 0)),
            pl.BlockSpec((_PB, 128), lambda i: (i, 0)),
            pl.BlockSpec((128, 128), lambda i: (0, 0)),
            pl.BlockSpec((128, 128), lambda i: (0, 0)),
            pl.BlockSpec((128, 128), lambda i: (0, 0)),
            pl.BlockSpec((128, 128), lambda i: (0, 0)),
            pl.BlockSpec((8, 128), lambda i: (0, 0)),
        ],
        out_specs=pl.BlockSpec((24, 16), lambda i: (0, 0)),
    )(g1v, table1, s3, d0, wx, wg, brow)


def _pairfold(z):
    z3 = z.reshape(z.shape[0] // 2, 2, 128)
    return z3[:, 0, :] + z3[:, 1, :]


def _groupsum(y):
    y = y + jnp.roll(y, 16, axis=1)
    y = y + jnp.roll(y, 32, axis=1)
    y = y + jnp.roll(y, 64, axis=1)
    return y


def _stage2_kernel(g1_ref, tab_ref, s3_ref, d0_ref, wx_ref, wg_ref, brow_ref,
                   wfc_ref, sg_ref, agg_ref, fx_ref, mom_ref):
    g1 = g1_ref[...]
    fxyz = jnp.maximum(
        _dis_and_terms(g1, tab_ref[...], s3_ref, d0_ref, wx_ref, wg_ref,
                       brow_ref), 0.0)
    lane = _lanes_iota()
    nbmask = jnp.logical_and(lane >= 3, lane < 11)
    fcat = jnp.where(nbmask, g1, 0.0) + fxyz
    att = jnp.dot(fcat, wfc_ref[...], preferred_element_type=jnp.float32, precision=_PREC)
    e = jnp.exp(att)
    denom = _pairfold(_groupsum(e))
    numer = _pairfold(_groupsum(fcat * e))
    agg = numer / denom
    agg_ref[...] = agg
    fx_ref[...] = fxyz

    ca = lax.dot_general(agg, agg, (((0,), (0,)), ((), ())),
                         preferred_element_type=jnp.float32)
    magg = ca[0:16, 0:16]
    sagg = jnp.sum(agg, axis=0, keepdims=True)[:, 0:16]
    cx = lax.dot_general(fxyz, fxyz, (((0,), (0,)), ((), ())),
                         preferred_element_type=jnp.float32)
    mfx = jnp.zeros((16, 16), jnp.float32)
    for eb in range(8):
        mfx = mfx + cx[eb * 16:(eb + 1) * 16, eb * 16:(eb + 1) * 16]
    sx = jnp.sum(fxyz, axis=0, keepdims=True)
    sfx = jnp.zeros((1, 16), jnp.float32)
    for eb in range(8):
        sfx = sfx + sx[:, eb * 16:(eb + 1) * 16]
    blk = jnp.concatenate(
        [magg, sagg, jnp.zeros((7, 16), jnp.float32),
         mfx, sfx, jnp.zeros((7, 16), jnp.float32)], axis=0)

    @pl.when(pl.program_id(0) == 0)
    def _():
        mom_ref[...] = jnp.zeros_like(mom_ref)

    mom_ref[...] += blk


def _stage2(g1v, table1, s3, d0, wx, wg, brow, wfc, sg):
    return pl.pallas_call(
        _stage2_kernel,
        out_shape=[
            jax.ShapeDtypeStruct((_P, 128), jnp.float32),
            jax.ShapeDtypeStruct((_M // 8, 128), jnp.float32),
            jax.ShapeDtypeStruct((48, 16), jnp.float32),
        ],
        grid=(_NBLK,),
        in_specs=[
            pl.BlockSpec((_RB, 128), lambda i: (i, 0)),
            pl.BlockSpec((_PB, 128), lambda i: (i, 0)),
            pl.BlockSpec((128, 128), lambda i: (0, 0)),
            pl.BlockSpec((128, 128), lambda i: (0, 0)),
            pl.BlockSpec((128, 128), lambda i: (0, 0)),
            pl.BlockSpec((128, 128), lambda i: (0, 0)),
            pl.BlockSpec((8, 128), lambda i: (0, 0)),
            pl.BlockSpec((128, 128), lambda i: (0, 0)),
            pl.BlockSpec((128, 128), lambda i: (0, 0)),
        ],
        out_specs=[
            pl.BlockSpec((_PB, 128), lambda i: (i, 0)),
            pl.BlockSpec((_RB, 128), lambda i: (i, 0)),
            pl.BlockSpec((48, 16), lambda i: (0, 0)),
        ],
    )(g1v, table1, s3, d0, wx, wg, brow, wfc, sg)


def _stage3_kernel(g2_ref, fx_ref, w3_ref, b3_ref, w4_ref, b4_ref,
                   wfc_ref, sg_ref, agg_ref, mom_ref):
    fnb2 = jnp.maximum(
        jnp.dot(g2_ref[...], w3_ref[...], preferred_element_type=jnp.float32, precision=_PREC)
        + b3_ref[0:1, :], 0.0)
    fx2 = jnp.maximum(
        jnp.dot(fx_ref[...], w4_ref[...], preferred_element_type=jnp.float32, precision=_PREC)
        + b4_ref[0:1, :], 0.0)
    fcat = fnb2 + fx2
    att = jnp.dot(fcat, wfc_ref[...], preferred_element_type=jnp.float32, precision=_PREC)
    e = jnp.exp(att)
    denom = _pairfold(_groupsum(e))
    numer = _pairfold(_groupsum(fcat * e))
    agg = numer / denom
    agg_ref[...] = agg

    ca = lax.dot_general(agg, agg, (((0,), (0,)), ((), ())),
                         preferred_element_type=jnp.float32)
    blk = jnp.concatenate(
        [ca[0:16, 0:16], jnp.sum(agg, axis=0, keepdims=True)[:, 0:16],
         jnp.zeros((7, 16), jnp.float32)], axis=0)

    @pl.when(pl.program_id(0) == 0)
    def _():
        mom_ref[...] = jnp.zeros_like(mom_ref)

    mom_ref[...] += blk


def _stage3(g2v, fxtab, w3, b3, w4, b4, wfc, sg):
    return pl.pallas_call(
        _stage3_kernel,
        out_shape=[
            jax.ShapeDtypeStruct((_P, 128), jnp.float32),
            jax.ShapeDtypeStruct((24, 16), jnp.float32),
        ],
        grid=(_NBLK,),
        in_specs=[
            pl.BlockSpec((_RB, 128), lambda i: (i, 0)),
            pl.BlockSpec((_RB, 128), lambda i: (i, 0)),
            pl.BlockSpec((128, 128), lambda i: (0, 0)),
            pl.BlockSpec((8, 128), lambda i: (0, 0)),
            pl.BlockSpec((128, 128), lambda i: (0, 0)),
            pl.BlockSpec((8, 128), lambda i: (0, 0)),
            pl.BlockSpec((128, 128), lambda i: (0, 0)),
            pl.BlockSpec((128, 128), lambda i: (0, 0)),
        ],
        out_specs=[
            pl.BlockSpec((_PB, 128), lambda i: (i, 0)),
            pl.BlockSpec((24, 16), lambda i: (0, 0)),
        ],
    )(g2v, fxtab, w3, b3, w4, b4, wfc, sg)


def _fpc2_kernel(agg_ref, wp_ref, out_ref, mom_ref):
    a = agg_ref[...][:, 0:16]
    wp = wp_ref[...]
    f = jnp.maximum(
        jnp.dot(a, wp[0:16, :], preferred_element_type=jnp.float32, precision=_PREC)
        + wp[16:17, :], 0.0)
    out_ref[...] = f
    aa = jnp.concatenate(
        [f, jnp.ones((_FB, 1), jnp.float32), jnp.zeros((_FB, 15), jnp.float32)],
        axis=1)
    m = lax.dot_general(aa, aa, (((0,), (0,)), ((), ())),
                        preferred_element_type=jnp.float32)

    @pl.when(pl.program_id(0) == 0)
    def _():
        mom_ref[...] = jnp.zeros_like(mom_ref)

    mom_ref[...] += m


def _fpc2(agg2tab, wpack):
    return pl.pallas_call(
        _fpc2_kernel,
        out_shape=[
            jax.ShapeDtypeStruct((_P, 16), jnp.float32),
            jax.ShapeDtypeStruct((32, 32), jnp.float32),
        ],
        grid=(_NFB,),
        in_specs=[
            pl.BlockSpec((_FB, 128), lambda j: (j, 0)),
            pl.BlockSpec((24, 16), lambda j: (0, 0)),
        ],
        out_specs=[
            pl.BlockSpec((_FB, 16), lambda j: (j, 0)),
            pl.BlockSpec((32, 32), lambda j: (0, 0)),
        ],
    )(agg2tab, wpack)


def _final_kernel(fpc2_ref, ftr_ref, w6_ref, wsc_ref, out_ref):
    w6 = w6_ref[...]
    wsc = wsc_ref[...]
    y = (jnp.dot(fpc2_ref[...], w6[0:16, :], preferred_element_type=jnp.float32, precision=_PREC)
         + w6[16:17, :]
         + jnp.dot(ftr_ref[...], wsc[0:8, :], preferred_element_type=jnp.float32, precision=_PREC)
         + wsc[8:9, :])
    out_ref[...] = jnp.where(y >= 0.0, y, 0.2 * y)


def _final(fpc2, ftr, w6pack, wscpack):
    return pl.pallas_call(
        _final_kernel,
        out_shape=jax.ShapeDtypeStruct((_P, 32), jnp.float32),
        grid=(_NFB,),
        in_specs=[
            pl.BlockSpec((_FB, 16), lambda j: (j, 0)),
            pl.BlockSpec((_FB, 8), lambda j: (j, 0)),
            pl.BlockSpec((24, 32), lambda j: (0, 0)),
            pl.BlockSpec((16, 32), lambda j: (0, 0)),
        ],
        out_specs=pl.BlockSpec((_FB, 32), lambda j: (j, 0)),
    )(fpc2, ftr, w6pack, wscpack)


def _np_zeros(*s):
    return np.zeros(s, np.float32)


def _build_s3():
    m = _np_zeros(128, 128)
    for e in range(8):
        for j in range(3):
            m[e * 16 + j, e * 16] = 1.0
    return jnp.asarray(m)


def _build_sg():
    m = _np_zeros(128, 128)
    for e in range(8):
        for ep in range(8):
            for s in range(16):
                m[e * 16 + s, ep * 16 + s] = 1.0
    return jnp.asarray(m)


def _build_relpos_consts(W2, b2, s2, t2, out_slots):
    W2s = W2 * s2[:, None]
    b2s = b2 * s2 + t2
    idx_out = np.asarray(out_slots)
    r3 = np.arange(3)
    d0b = jnp.zeros((16, 16), jnp.float32).at[0, idx_out].set(W2s[:, 0])
    wxb = jnp.zeros((16, 16), jnp.float32).at[np.ix_(r3, idx_out)].set(
        (W2s[:, 4:7] + W2s[:, 1:4]).T)
    wgb = jnp.zeros((16, 16), jnp.float32).at[np.ix_(r3, idx_out)].set(
        (W2s[:, 7:10] - W2s[:, 1:4]).T)
    bvec = jnp.zeros((16,), jnp.float32).at[idx_out].set(b2s)
    return (_bdiag_j(d0b), _bdiag_j(wxb), _bdiag_j(wgb), _rowpat(bvec))


def _fcat1_ch_of_slot():
    ch = [0] * 16
    for j in range(8):
        ch[3 + j] = j
    for j, s in enumerate(_XSLOT):
        ch[s] = 8 + j
    return ch


def kernel(feature, xyz, neigh_idx, params):
    p = params
    lfa = p['lfa']
    ftr = jnp.transpose(feature[..., 0], (0, 2, 1)).reshape(_P, 8)
    xyzf = xyz.reshape(_P, 3)

    momf = _mom_feat(ftr)
    s1f, m2f = momf[8, 0:8], momf[0:8, 0:8]
    W1f, b1f = _affine_from_in_moments(
        p['mlp1']['W'], p['mlp1']['b'], p['mlp1']['g'], p['mlp1']['be'],
        s1f, m2f, _P)
    Wscf, bscf = _affine_from_in_moments(
        p['shortcut']['W'], p['shortcut']['b'], p['shortcut']['g'],
        p['shortcut']['be'], s1f, m2f, _P)

    wpack1 = jnp.zeros((16, 16), jnp.float32)
    wpack1 = wpack1.at[0:8, 0:8].set(W1f.T).at[8, 0:8].set(b1f)
    table1 = _table1(ftr, xyzf, wpack1)

    g1v = _gatherv(table1, neigh_idx)

    s3 = _build_s3()
    sg = _build_sg()
    W2, b2 = lfa['mlp1']['W'], lfa['mlp1']['b']
    one8 = jnp.ones((8,), jnp.float32)
    zero8 = jnp.zeros((8,), jnp.float32)
    d0r, wxr, wgr, browr = _build_relpos_consts(
        W2, b2, one8, zero8, tuple(range(8)))
    mom2 = _mom2(g1v, table1, s3, d0r, wxr, wgr, browr)
    m2y, s2y = mom2[0:8, 0:8], mom2[16, 0:8]
    s2, t2 = _affine_from_out_moments(
        lfa['mlp1']['g'], lfa['mlp1']['be'], s2y, jnp.diag(m2y), _M)

    d0, wx, wg, brow = _build_relpos_consts(W2, b2, s2, t2, _XSLOT)
    ch1 = _fcat1_ch_of_slot()
    slot_of_ch1 = [0] * 16
    for s in range(16):
        slot_of_ch1[ch1[s]] = s
    sm1 = _slot_matrix_j(lfa['att1']['Wfc'], ch1, slot_of_ch1)
    wfc1 = _bdiag_j(sm1)
    aggtab, fxtab, mom34 = _stage2(
        g1v, table1, s3, d0, wx, wg, brow, wfc1, sg)
    magg, sagg = mom34[0:16, 0:16], mom34[16, 0:16]
    mfx, sfx = mom34[24:40, 0:16], mom34[40, 0:16]

    idx1 = jnp.asarray(slot_of_ch1)
    magg_c = magg[jnp.ix_(idx1, idx1)]
    sagg_c = sagg[idx1]
    W3f, b3f = _affine_from_in_moments(
        lfa['att1']['mlp']['W'], lfa['att1']['mlp']['b'],
        lfa['att1']['mlp']['g'], lfa['att1']['mlp']['be'],
        sagg_c, magg_c, _P)
    w3slot = _slot_matrix_j(W3f, ch1, list(range(8)))
    w3bd = _bdiag_j(w3slot)
    b3row = _rowpat(jnp.zeros((16,), jnp.float32).at[0:8].set(b3f))

    idxx = jnp.asarray(list(_XSLOT))
    mfx_c = mfx[jnp.ix_(idxx, idxx)]
    sfx_c = sfx[idxx]
    W4f, b4f = _affine_from_in_moments(
        lfa['mlp2']['W'], lfa['mlp2']['b'], lfa['mlp2']['g'],
        lfa['mlp2']['be'], sfx_c, mfx_c, _M)
    w4slot = _w4_slot_matrix(W4f)
    w4bd = _bdiag_j(w4slot)
    b4row = _rowpat(jnp.zeros((16,), jnp.float32).at[8:16].set(b4f))

    g2v = _gatherv(aggtab, neigh_idx)

    ch2 = list(range(16))
    sm2 = _slot_matrix_j(lfa['att2']['Wfc'], ch2, list(range(16)))
    wfc2 = _bdiag_j(sm2)
    agg2tab, mom5 = _stage3(
        g2v, fxtab, w3bd, b3row, w4bd, b4row, wfc2, sg)
    magg2, sagg2 = mom5[0:16, 0:16], mom5[16, 0:16]

    W5f, b5f = _affine_from_in_moments(
        lfa['att2']['mlp']['W'], lfa['att2']['mlp']['b'],
        lfa['att2']['mlp']['g'], lfa['att2']['mlp']['be'],
        sagg2, magg2, _P)
    wpack5 = jnp.zeros((24, 16), jnp.float32)
    wpack5 = wpack5.at[0:16, :].set(W5f.T).at[16, :].set(b5f)
    fpc2, mom6 = _fpc2(agg2tab, wpack5)
    m6, s6 = mom6[0:16, 0:16], mom6[16, 0:16]
    mu6 = s6 / _P
    cov6 = m6 / _P - jnp.outer(mu6, mu6)
    W6, b6 = p['mlp2']['W'], p['mlp2']['b']
    mean6 = W6 @ mu6 + b6
    var6 = jnp.einsum('oi,ij,oj->o', W6, cov6, W6)
    sca6 = p['mlp2']['g'] / jnp.sqrt(var6 + _EPS)
    W6f = W6 * sca6[:, None]
    b6f = sca6 * (b6 - mean6) + p['mlp2']['be']

    w6pack = jnp.zeros((24, 32), jnp.float32)
    w6pack = w6pack.at[0:16, :].set(W6f.T).at[16, :].set(b6f)
    wscpack = jnp.zeros((16, 32), jnp.float32)
    wscpack = wscpack.at[0:8, :].set(Wscf.T).at[8, :].set(bscf)

    out = _final(fpc2, ftr, w6pack, wscpack)
    out = out.reshape(_B, _N, 32)
    return jnp.transpose(out, (0, 2, 1))[..., None]


def _slot_matrix_j(Wfc, ch_of_slot, out_list):
    Wfc = jnp.asarray(Wfc)
    nout = Wfc.shape[0]
    m = jnp.zeros((16, 16), jnp.float32)
    for si in range(16):
        for o in range(nout):
            m = m.at[si, out_list[o]].set(Wfc[o, ch_of_slot[si]])
    return m


def _w4_slot_matrix(W4f):
    m = jnp.zeros((16, 16), jnp.float32)
    for j in range(8):
        for o in range(8):
            m = m.at[_XSLOT[j], 8 + o].set(W4f[o, j])
    return m
